# baseline (speedup 1.0000x reference)
import functools

import jax
import jax.numpy as jnp
from jax import lax
from jax.experimental import pallas as pl
from jax.experimental.pallas import tpu as pltpu
from jax.experimental.pallas import tpu_sc as plsc

_ROWS_PER_TILE = 4

_mesh = plsc.VectorSubcoreMesh(core_axis_name="c", subcore_axis_name="s")


@functools.partial(
    pl.kernel,
    mesh=_mesh,
    out_type=jax.ShapeDtypeStruct((1, 2, 64, 128), jnp.float32),
    scratch_types=[
        pltpu.VMEM((16,), jnp.int32),
        pltpu.VMEM((_ROWS_PER_TILE, 128), jnp.float32),
    ],
)
def _extract(table_hbm, step_hbm, out_hbm, step_v, buf_v):
    wid = lax.axis_index("s") * 2 + lax.axis_index("c")
    c = wid // 16
    r0 = (wid % 16) * _ROWS_PER_TILE
    pltpu.sync_copy(step_hbm, step_v)
    s = step_v[...][0]
    pltpu.sync_copy(table_hbm.at[s, c, pl.ds(r0, _ROWS_PER_TILE)], buf_v)
    pltpu.sync_copy(buf_v, out_hbm.at[0, c, pl.ds(r0, _ROWS_PER_TILE)])


def kernel(table, step):
    step_vec = jnp.full((16,), step, dtype=jnp.int32)
    tview = jnp.swapaxes(table, 2, 3)
    out = _extract(tview, step_vec)
    return jnp.swapaxes(out, 2, 3)


# speedup vs baseline: 1.0863x; 1.0863x over previous
import functools

import jax
import jax.numpy as jnp
from jax import lax
from jax.experimental import pallas as pl
from jax.experimental.pallas import tpu as pltpu
from jax.experimental.pallas import tpu_sc as plsc

_mesh = plsc.ScalarSubcoreMesh(axis_name="c", num_cores=1)


@functools.partial(
    pl.kernel,
    mesh=_mesh,
    out_type=jax.ShapeDtypeStruct((1, 2, 64, 128), jnp.float32),
    scratch_types=[
        pltpu.SMEM((1,), jnp.int32),
    ],
)
def _extract(table_hbm, step_hbm, out_hbm, step_s):
    pltpu.sync_copy(step_hbm, step_s)
    s = step_s[0]
    pltpu.sync_copy(table_hbm.at[pl.ds(s, 1)], out_hbm)


def kernel(table, step):
    step_vec = jnp.reshape(jnp.asarray(step, dtype=jnp.int32), (1,))
    tview = jnp.swapaxes(table, 2, 3)
    out = _extract(tview, step_vec)
    return jnp.swapaxes(out, 2, 3)


# speedup vs baseline: 1.1714x; 1.0783x over previous
import functools

import jax
import jax.numpy as jnp
from jax import lax
from jax.experimental import pallas as pl
from jax.experimental.pallas import tpu as pltpu
from jax.experimental.pallas import tpu_sc as plsc

_mesh = plsc.ScalarSubcoreMesh(axis_name="c", num_cores=1)


@functools.partial(
    pl.kernel,
    mesh=_mesh,
    out_type=jax.ShapeDtypeStruct((1, 2, 64, 128), jnp.float32),
    scratch_types=[
        pltpu.SMEM((1,), jnp.int32),
        pltpu.VMEM_SHARED((1, 2, 64, 128), jnp.float32),
    ],
)
def _extract(table_hbm, step_hbm, out_hbm, step_s, row_sp):
    pltpu.sync_copy(step_hbm, step_s)
    s = step_s[0]
    pltpu.sync_copy(table_hbm.at[pl.ds(s, 1)], row_sp)
    pltpu.sync_copy(row_sp, out_hbm)


def kernel(table, step):
    step_vec = jnp.reshape(jnp.asarray(step, dtype=jnp.int32), (1,))
    tview = jnp.swapaxes(table, 2, 3)
    out = _extract(tview, step_vec)
    return jnp.swapaxes(out, 2, 3)


# speedup vs baseline: 1.1830x; 1.0100x over previous
import functools

import jax
import jax.numpy as jnp
from jax import lax
from jax.experimental import pallas as pl
from jax.experimental.pallas import tpu as pltpu
from jax.experimental.pallas import tpu_sc as plsc

_mesh = plsc.ScalarSubcoreMesh(axis_name="c", num_cores=1)


@functools.partial(
    pl.kernel,
    mesh=_mesh,
    out_type=jax.ShapeDtypeStruct((1, 2, 64, 128), jnp.float32),
    scratch_types=[
        pltpu.SMEM((1,), jnp.int32),
        pltpu.VMEM_SHARED((1, 2, 64, 128), jnp.float32),
        pltpu.SemaphoreType.DMA,
        pltpu.SemaphoreType.DMA,
        pltpu.SemaphoreType.DMA,
        pltpu.SemaphoreType.DMA,
    ],
)
def _extract(table_hbm, step_hbm, out_hbm, step_s, row_sp, si0, si1, so0, so1):
    pltpu.sync_copy(step_hbm, step_s)
    s = step_s[0]
    in0 = pltpu.make_async_copy(
        table_hbm.at[pl.ds(s, 1), pl.ds(0, 1)], row_sp.at[:, pl.ds(0, 1)], si0)
    in1 = pltpu.make_async_copy(
        table_hbm.at[pl.ds(s, 1), pl.ds(1, 1)], row_sp.at[:, pl.ds(1, 1)], si1)
    in0.start()
    in1.start()
    in0.wait()
    out0 = pltpu.make_async_copy(
        row_sp.at[:, pl.ds(0, 1)], out_hbm.at[:, pl.ds(0, 1)], so0)
    out0.start()
    in1.wait()
    out1 = pltpu.make_async_copy(
        row_sp.at[:, pl.ds(1, 1)], out_hbm.at[:, pl.ds(1, 1)], so1)
    out1.start()
    out0.wait()
    out1.wait()


def kernel(table, step):
    step_vec = jnp.reshape(jnp.asarray(step, dtype=jnp.int32), (1,))
    tview = jnp.swapaxes(table, 2, 3)
    out = _extract(tview, step_vec)
    return jnp.swapaxes(out, 2, 3)
